# baseline (speedup 1.0000x reference)
import jax
import jax.numpy as jnp
from jax import lax
from jax.experimental import pallas as pl
from jax.experimental.pallas import tpu as pltpu
from jax.experimental.pallas import tpu_sc as plsc

EMB = 32
BATCH = 16384
NC = 2
NS = 16
NW = NC * NS
RPW = BATCH // NW

CH = 4
DEPTH = 3
NCHUNKS = RPW // CH

BCHUNK = 128
BNCH = RPW // BCHUNK
IDS_MAJOR = BATCH // BCHUNK


def _sc_uq_body(uids, iids, u_tab, q_tab, b_tab,
                u_out, q_out, b_out,
                su, si, b_rows, ublk, qblk, ustage, qstage, sem, bsem):
    w = lax.axis_index("s") * NC + lax.axis_index("c")
    base = w * RPW
    rows = lax.iota(jnp.int32, 16)

    def fire(idu, idq, j, slot):
        uid = idu[j]
        iid = idq[j]
        cu = pl.multiple_of((uid // 128) * 128, 128)
        cq = pl.multiple_of((iid // 128) * 128, 128)
        return [
            pltpu.async_copy(u_tab.at[:, pl.ds(cu, 128)], ublk.at[slot], sem),
            pltpu.async_copy(q_tab.at[:, pl.ds(cq, 128)], qblk.at[slot], sem),
        ]

    def extract(idu, idq, c0, j, slot):
        uid = idu[j]
        iid = idq[j]
        lu = jnp.zeros((16,), jnp.int32) + (uid % 128)
        lq = jnp.zeros((16,), jnp.int32) + (iid % 128)
        col = jnp.zeros((16,), jnp.int32) + (c0 + j)
        plsc.store_scatter(ustage, [rows, col],
                           plsc.load_gather(ublk.at[slot], [rows, lu]))
        plsc.store_scatter(ustage, [rows + 16, col],
                           plsc.load_gather(ublk.at[slot], [rows + 16, lu]))
        plsc.store_scatter(qstage, [rows, col],
                           plsc.load_gather(qblk.at[slot], [rows, lq]))
        plsc.store_scatter(qstage, [rows + 16, col],
                           plsc.load_gather(qblk.at[slot], [rows + 16, lq]))

    NWAVE = 16 // CH

    HALF = RPW // 2

    def chunk(ci, _):
        i0 = ci * 16
        idu = su[pl.ds(i0, 16)]
        idq = si[pl.ds(i0, 16)]
        inflight = []
        for g in range(NWAVE):
            cur = []
            for j4 in range(CH):
                cur += fire(idu, idq, g * CH + j4, (g % DEPTH) * CH + j4)
            inflight.append((g, cur))
            if len(inflight) == DEPTH:
                gd, cs = inflight.pop(0)
                for c in cs:
                    c.wait()
                for j4 in range(CH):
                    extract(idu, idq, i0, gd * CH + j4,
                            (gd % DEPTH) * CH + j4)
        for gd, cs in inflight:
            for c in cs:
                c.wait()
            for j4 in range(CH):
                extract(idu, idq, i0, gd * CH + j4, (gd % DEPTH) * CH + j4)
        return ()

    for h in range(2):
        pltpu.sync_copy(uids.at[pl.ds(base + h * HALF, HALF)], su)
        pltpu.sync_copy(iids.at[pl.ds(base + h * HALF, HALF)], si)
        bcopies = [
            pltpu.async_copy(b_tab.at[si.at[pl.ds(k * BCHUNK, BCHUNK)]],
                             b_rows.at[h * 2 + k], bsem)
            for k in range(2)
        ]
        lax.fori_loop(0, HALF // 16, chunk, (), unroll=False)
        pltpu.sync_copy(ustage, u_out.at[:, pl.ds(base + h * HALF, HALF)])
        pltpu.sync_copy(qstage, q_out.at[:, pl.ds(base + h * HALF, HALF)])
        for c in bcopies:
            c.wait()
    for j in range(BNCH):
        pltpu.sync_copy(b_rows.at[j], b_out.at[pl.ds(base + j * BCHUNK, BCHUNK)])


def _make_sc_uq():
    return pl.kernel(
        _sc_uq_body,
        out_type=[
            jax.ShapeDtypeStruct((EMB, BATCH), jnp.float32),
            jax.ShapeDtypeStruct((EMB, BATCH), jnp.float32),
            jax.ShapeDtypeStruct((BATCH,), jnp.float32),
        ],
        mesh=plsc.VectorSubcoreMesh(core_axis_name="c", subcore_axis_name="s"),
        compiler_params=pltpu.CompilerParams(
            use_tc_tiling_on_sc=True, needs_layout_passes=False),
        scratch_types=[
            pltpu.VMEM((RPW // 2,), jnp.int32),
            pltpu.VMEM((RPW // 2,), jnp.int32),
            pltpu.VMEM((BNCH, BCHUNK), jnp.float32),
            pltpu.VMEM((DEPTH * CH, EMB, 128), jnp.float32),
            pltpu.VMEM((DEPTH * CH, EMB, 128), jnp.float32),
            pltpu.VMEM((EMB, RPW // 2), jnp.float32),
            pltpu.VMEM((EMB, RPW // 2), jnp.float32),
            pltpu.SemaphoreType.DMA,
            pltpu.SemaphoreType.DMA,
        ],
    )


BLK = 2048
GRID = BATCH // BLK


def _tc_body(u_ref, q_ref, b_ref, w1t_ref, b1_ref, w2_ref, b2_ref,
             pred_ref, score_ref):
    u = u_ref[...]
    q = q_ref[...]
    uq = u * q
    reg = jnp.concatenate([u, q, uq], axis=0)
    h = lax.dot_general(w1t_ref[...], reg, (((0,), (0,)), ((), ())),
                        preferred_element_type=jnp.float32)
    h = jnp.maximum(h + b1_ref[...], 0.0)
    score = jnp.sum(h * w2_ref[...], axis=0) + b2_ref[0, 0]
    fact = jnp.sum(uq, axis=0)
    pred_ref[...] = fact + b_ref[...]
    score_ref[...] = score


def _tc_call(u2, q2, bg, W1t, b1r, W2, b2r):
    return pl.pallas_call(
        _tc_body,
        grid=(GRID,),
        in_specs=[
            pl.BlockSpec((EMB, BLK), lambda i: (0, i)),
            pl.BlockSpec((EMB, BLK), lambda i: (0, i)),
            pl.BlockSpec((BLK,), lambda i: (i,)),
            pl.BlockSpec((3 * EMB, 64), lambda i: (0, 0)),
            pl.BlockSpec((64, 1), lambda i: (0, 0)),
            pl.BlockSpec((64, 1), lambda i: (0, 0)),
            pl.BlockSpec((1, 1), lambda i: (0, 0)),
        ],
        out_specs=[
            pl.BlockSpec((BLK,), lambda i: (i,)),
            pl.BlockSpec((BLK,), lambda i: (i,)),
        ],
        out_shape=[
            jax.ShapeDtypeStruct((BATCH,), jnp.float32),
            jax.ShapeDtypeStruct((BATCH,), jnp.float32),
        ],
    )(u2, q2, bg, W1t, b1r, W2, b2r)


def kernel(user_ids, item_ids, U, Q, B, W1, b1, W2, b2):
    uids = user_ids.astype(jnp.int32)
    iids = item_ids.astype(jnp.int32)
    u2, q2, b_g = _make_sc_uq()(uids, iids, U.T, Q.T, B.reshape(-1))
    pred, score = _tc_call(u2, q2, b_g, W1, b1.reshape(64, 1),
                           W2, b2.reshape(1, 1))
    return pred, score


# speedup vs baseline: 1.1289x; 1.1289x over previous
import jax
import jax.numpy as jnp
from jax import lax
from jax.experimental import pallas as pl
from jax.experimental.pallas import tpu as pltpu
from jax.experimental.pallas import tpu_sc as plsc

EMB = 32
BATCH = 16384
NC = 2
NS = 16
NW = NC * NS
RPW = BATCH // NW

CH = 4
DEPTH = 2
NCHUNKS = RPW // CH

BCHUNK = 128
BNCH = RPW // BCHUNK
IDS_MAJOR = BATCH // BCHUNK


def _sc_uq_body(uids, iids, u_tab, q_tab, b_tab,
                u_out, q_out, b_out,
                su, si, b_rows, ublk, qblk, ustage, qstage, sem, bsem):
    w = lax.axis_index("s") * NC + lax.axis_index("c")
    base = w * RPW
    rows = lax.iota(jnp.int32, 16)

    def fire(idu, idq, j, slot):
        uid = idu[j]
        iid = idq[j]
        cu = pl.multiple_of((uid // 128) * 128, 128)
        cq = pl.multiple_of((iid // 128) * 128, 128)
        return [
            pltpu.async_copy(u_tab.at[:, pl.ds(cu, 128)], ublk.at[slot], sem),
            pltpu.async_copy(q_tab.at[:, pl.ds(cq, 128)], qblk.at[slot], sem),
        ]

    def extract(idu, idq, c0, j, slot):
        uid = idu[j]
        iid = idq[j]
        lu = jnp.zeros((16,), jnp.int32) + (uid % 128)
        lq = jnp.zeros((16,), jnp.int32) + (iid % 128)
        col = jnp.zeros((16,), jnp.int32) + (c0 + j)
        plsc.store_scatter(ustage, [rows, col],
                           plsc.load_gather(ublk.at[slot], [rows, lu]))
        plsc.store_scatter(ustage, [rows + 16, col],
                           plsc.load_gather(ublk.at[slot], [rows + 16, lu]))
        plsc.store_scatter(qstage, [rows, col],
                           plsc.load_gather(qblk.at[slot], [rows, lq]))
        plsc.store_scatter(qstage, [rows + 16, col],
                           plsc.load_gather(qblk.at[slot], [rows + 16, lq]))

    HALF = RPW // 2
    NWAVES = HALF // CH
    NGROUP = (NWAVES + 2 + 3) // 4

    def wait_slot(slot):
        pltpu.make_async_copy(
            u_tab.at[:, pl.ds(0, 128)], ublk.at[slot], sem).wait()
        pltpu.make_async_copy(
            q_tab.at[:, pl.ds(0, 128)], qblk.at[slot], sem).wait()

    def wgroup(g, _):
        ic = jnp.minimum(g, NWAVES // 4 - 1)
        iduc = su[pl.ds(ic * 16, 16)]
        idqc = si[pl.ds(ic * 16, 16)]
        ip = jnp.maximum(g - 1, 0)
        idup = su[pl.ds(ip * 16, 16)]
        idqp = si[pl.ds(ip * 16, 16)]
        for k in range(4):
            w2 = g * 4 + k - 2
            k2 = (k + 2) % 4
            idu2, idq2 = (idup, idqp) if k < 2 else (iduc, idqc)

            @pl.when((w2 >= 0) & (w2 < NWAVES))
            def _():
                for j in range(CH):
                    slot = (k2 % 2) * CH + j
                    wait_slot(slot)
                    extract(idu2, idq2, w2 * CH - k2 * CH,
                            k2 * CH + j, slot)

            @pl.when(g * 4 + k < NWAVES)
            def _():
                for j in range(CH):
                    fire(iduc, idqc, k * CH + j, (k % 2) * CH + j)
        return ()

    for h in range(2):
        pltpu.sync_copy(uids.at[pl.ds(base + h * HALF, HALF)], su)
        pltpu.sync_copy(iids.at[pl.ds(base + h * HALF, HALF)], si)
        bcopies = [
            pltpu.async_copy(b_tab.at[si.at[pl.ds(k * BCHUNK, BCHUNK)]],
                             b_rows.at[h * 2 + k], bsem)
            for k in range(2)
        ]
        lax.fori_loop(0, NGROUP, wgroup, (), unroll=False)
        pltpu.sync_copy(ustage, u_out.at[:, pl.ds(base + h * HALF, HALF)])
        pltpu.sync_copy(qstage, q_out.at[:, pl.ds(base + h * HALF, HALF)])
        for c in bcopies:
            c.wait()
    for j in range(BNCH):
        pltpu.sync_copy(b_rows.at[j], b_out.at[pl.ds(base + j * BCHUNK, BCHUNK)])


def _make_sc_uq():
    return pl.kernel(
        _sc_uq_body,
        out_type=[
            jax.ShapeDtypeStruct((EMB, BATCH), jnp.float32),
            jax.ShapeDtypeStruct((EMB, BATCH), jnp.float32),
            jax.ShapeDtypeStruct((BATCH,), jnp.float32),
        ],
        mesh=plsc.VectorSubcoreMesh(core_axis_name="c", subcore_axis_name="s"),
        compiler_params=pltpu.CompilerParams(
            use_tc_tiling_on_sc=True, needs_layout_passes=False),
        scratch_types=[
            pltpu.VMEM((RPW // 2,), jnp.int32),
            pltpu.VMEM((RPW // 2,), jnp.int32),
            pltpu.VMEM((BNCH, BCHUNK), jnp.float32),
            pltpu.VMEM((DEPTH * CH, EMB, 128), jnp.float32),
            pltpu.VMEM((DEPTH * CH, EMB, 128), jnp.float32),
            pltpu.VMEM((EMB, RPW // 2), jnp.float32),
            pltpu.VMEM((EMB, RPW // 2), jnp.float32),
            pltpu.SemaphoreType.DMA,
            pltpu.SemaphoreType.DMA,
        ],
    )


BLK = 2048
GRID = BATCH // BLK


def _tc_body(u_ref, q_ref, b_ref, w1t_ref, b1_ref, w2_ref, b2_ref,
             pred_ref, score_ref):
    u = u_ref[...]
    q = q_ref[...]
    uq = u * q
    reg = jnp.concatenate([u, q, uq], axis=0)
    h = lax.dot_general(w1t_ref[...], reg, (((0,), (0,)), ((), ())),
                        preferred_element_type=jnp.float32)
    h = jnp.maximum(h + b1_ref[...], 0.0)
    score = jnp.sum(h * w2_ref[...], axis=0) + b2_ref[0, 0]
    fact = jnp.sum(uq, axis=0)
    pred_ref[...] = fact + b_ref[...]
    score_ref[...] = score


def _tc_call(u2, q2, bg, W1t, b1r, W2, b2r):
    return pl.pallas_call(
        _tc_body,
        grid=(GRID,),
        in_specs=[
            pl.BlockSpec((EMB, BLK), lambda i: (0, i)),
            pl.BlockSpec((EMB, BLK), lambda i: (0, i)),
            pl.BlockSpec((BLK,), lambda i: (i,)),
            pl.BlockSpec((3 * EMB, 64), lambda i: (0, 0)),
            pl.BlockSpec((64, 1), lambda i: (0, 0)),
            pl.BlockSpec((64, 1), lambda i: (0, 0)),
            pl.BlockSpec((1, 1), lambda i: (0, 0)),
        ],
        out_specs=[
            pl.BlockSpec((BLK,), lambda i: (i,)),
            pl.BlockSpec((BLK,), lambda i: (i,)),
        ],
        out_shape=[
            jax.ShapeDtypeStruct((BATCH,), jnp.float32),
            jax.ShapeDtypeStruct((BATCH,), jnp.float32),
        ],
    )(u2, q2, bg, W1t, b1r, W2, b2r)


def kernel(user_ids, item_ids, U, Q, B, W1, b1, W2, b2):
    uids = user_ids.astype(jnp.int32)
    iids = item_ids.astype(jnp.int32)
    u2, q2, b_g = _make_sc_uq()(uids, iids, U.T, Q.T, B.reshape(-1))
    pred, score = _tc_call(u2, q2, b_g, W1, b1.reshape(64, 1),
                           W2, b2.reshape(1, 1))
    return pred, score
